# Initial kernel scaffold; baseline (speedup 1.0000x reference)
import jax
import jax.numpy as jnp
from jax.experimental import pallas as pl


def kernel(h, edge_index, W, b):
    raise NotImplementedError("write your pallas kernel here")


# speedup vs baseline: 3.7553x; 3.7553x over previous
import functools

import jax
import jax.numpy as jnp
from jax import lax
from jax.experimental import pallas as pl
from jax.experimental.pallas import tpu as pltpu
from jax.experimental.pallas import tpu_sc as plsc

N = 10000
F = 128
E = 320000
NC = 2
NS = 16
NW = NC * NS
CH = 128
SUB = 40
NSUP = 2
NCH = SUB * NSUP
E_PW = NCH * CH
E_PAD = NW * E_PW
NBUF = 2
ROWS_PT = 640
N_PAD = NS * ROWS_PT


def _mm_body(h_ref, w_ref, o_ref):
    o_ref[...] = jnp.dot(h_ref[...], w_ref[...],
                         preferred_element_type=jnp.float32)


def _matmul(h, W):
    return pl.pallas_call(
        _mm_body,
        grid=(10,),
        in_specs=[
            pl.BlockSpec((N // 10, F), lambda i: (i, 0)),
            pl.BlockSpec((F, F), lambda i: (0, 0)),
        ],
        out_specs=pl.BlockSpec((N // 10, F), lambda i: (i, 0)),
        out_shape=jax.ShapeDtypeStruct((N, F), jnp.float32),
    )(h, W)


_MESH = plsc.VectorSubcoreMesh(core_axis_name="c", subcore_axis_name="s")


@functools.partial(
    pl.kernel,
    out_type=jax.ShapeDtypeStruct((NC, N_PAD, F), jnp.float32),
    mesh=_MESH,
    scratch_types=[
        pltpu.VMEM((2, SUB, CH), jnp.int32),
        pltpu.VMEM((CH, F), jnp.float32),
        pltpu.VMEM((CH, F), jnp.float32),
        pltpu.VMEM_SHARED((N_PAD, F), jnp.float32),
        pltpu.SemaphoreType.DMA,
        pltpu.SemaphoreType.DMA,
    ],
)
def _scatter_gather(edges_hbm, hw_hbm, zeros_hbm, out_hbm,
                    idx_win, rows0, rows1, accum, sem0, sem1):
    c = lax.axis_index("c")
    s = lax.axis_index("s")
    wid = c * NS + s
    base = s * ROWS_PT
    bufs = (rows0, rows1)
    sems = (sem0, sem1)

    def wait_gather(b):
        pltpu.make_async_copy(hw_hbm.at[idx_win.at[0, 0]],
                              bufs[b], sems[b]).wait()

    pltpu.sync_copy(zeros_hbm, accum.at[pl.ds(base, ROWS_PT)])
    plsc.subcore_barrier()

    for sup in range(NSUP):
        pltpu.sync_copy(edges_hbm.at[wid, sup], idx_win)

        for b in range(NBUF):
            pltpu.async_copy(hw_hbm.at[idx_win.at[0, b]], bufs[b], sems[b])

        def chunk_step(it, carry):
            g = it * NBUF
            for b in range(NBUF):
                j = g + b
                wait_gather(b)
                pltpu.sync_copy(bufs[b], accum.at[idx_win.at[1, j]],
                                add=True)
                pltpu.async_copy(hw_hbm.at[idx_win.at[0, j + NBUF]],
                                 bufs[b], sems[b])
            return carry

        lax.fori_loop(0, (SUB - NBUF) // NBUF, chunk_step, 0)

        for b in range(NBUF):
            j = SUB - NBUF + b
            wait_gather(b)
            pltpu.sync_copy(bufs[b], accum.at[idx_win.at[1, j]], add=True)

    plsc.subcore_barrier()

    pltpu.sync_copy(accum.at[pl.ds(base, ROWS_PT)],
                    out_hbm.at[c, pl.ds(base, ROWS_PT)])


def _ep_body(p_ref, b_ref, o_ref):
    o_ref[...] = jnp.maximum(p_ref[0] + p_ref[1] + b_ref[...], 0.0)


def _epilogue(partials, b):
    return pl.pallas_call(
        _ep_body,
        grid=(10,),
        in_specs=[
            pl.BlockSpec((NC, N // 10, F), lambda i: (0, i, 0)),
            pl.BlockSpec((1, F), lambda i: (0, 0)),
        ],
        out_specs=pl.BlockSpec((N // 10, F), lambda i: (i, 0)),
        out_shape=jax.ShapeDtypeStruct((N, F), jnp.float32),
    )(partials, b.reshape(1, F))


def kernel(h, edge_index, W, b):
    e = edge_index.astype(jnp.int32)
    pad = E_PAD - E
    src = jnp.concatenate([e[0], jnp.zeros((pad,), jnp.int32)])
    dst = jnp.concatenate([e[1], jnp.full((pad,), N, jnp.int32)])
    src = src.reshape(NW, NSUP, SUB, CH)
    dst = dst.reshape(NW, NSUP, SUB, CH)
    edges = jnp.stack([src, dst], axis=2)
    zeros = jnp.zeros((ROWS_PT, F), jnp.float32)

    hw = _matmul(h, W)
    partials = _scatter_gather(edges, hw, zeros)
    return _epilogue(partials, b)


# speedup vs baseline: 11.9003x; 3.1689x over previous
import functools

import jax
import jax.numpy as jnp
from jax import lax
from jax.experimental import pallas as pl
from jax.experimental.pallas import tpu as pltpu
from jax.experimental.pallas import tpu_sc as plsc

N = 10000
F = 128
E = 320000
NC = 2
NS = 16
NW = NC * NS
CH = 128
SUB = 40
NSUP = 2
NCH = SUB * NSUP
E_PW = NCH * CH
E_PAD = NW * E_PW
NBUF = 2
ROWS_PT = 640
N_PAD = NS * ROWS_PT


def _mm_body(h_ref, w_ref, o_ref):
    o_ref[...] = jnp.dot(h_ref[...], w_ref[...],
                         preferred_element_type=jnp.float32)


def _matmul(h, W):
    return pl.pallas_call(
        _mm_body,
        grid=(10,),
        in_specs=[
            pl.BlockSpec((N // 10, F), lambda i: (i, 0)),
            pl.BlockSpec((F, F), lambda i: (0, 0)),
        ],
        out_specs=pl.BlockSpec((N // 10, F), lambda i: (i, 0)),
        out_shape=jax.ShapeDtypeStruct((N, F), jnp.float32),
    )(h, W)


_MESH = plsc.VectorSubcoreMesh(core_axis_name="c", subcore_axis_name="s")


@functools.partial(
    pl.kernel,
    out_type=jax.ShapeDtypeStruct((NC, N_PAD, F), jnp.float32),
    mesh=_MESH,
    scratch_types=[
        pltpu.VMEM((2, SUB, CH), jnp.int32),
        pltpu.VMEM((CH, F), jnp.float32),
        pltpu.VMEM((CH, F), jnp.float32),
        pltpu.VMEM_SHARED((N_PAD, F), jnp.float32),
        pltpu.SemaphoreType.DMA,
        pltpu.SemaphoreType.DMA,
    ],
)
def _scatter_gather(edges_hbm, hw_hbm, zeros_hbm, out_hbm,
                    idx_win, rows0, rows1, accum, sem0, sem1):
    c = lax.axis_index("c")
    s = lax.axis_index("s")
    wid = c * NS + s
    base = s * ROWS_PT
    bufs = (rows0, rows1)
    sems = (sem0, sem1)

    def wait_gather(b):
        pltpu.make_async_copy(hw_hbm.at[idx_win.at[0, 0]],
                              bufs[b], sems[b]).wait()

    pltpu.sync_copy(zeros_hbm, accum.at[pl.ds(base, ROWS_PT)])
    plsc.subcore_barrier()

    for sup in range(NSUP):
        pltpu.sync_copy(edges_hbm.at[wid, sup], idx_win)

        for b in range(NBUF):
            pltpu.async_copy(hw_hbm.at[idx_win.at[0, b]], bufs[b], sems[b])

        def chunk_step(it, carry):
            g = it * NBUF
            for b in range(NBUF):
                j = g + b
                wait_gather(b)
                pltpu.sync_copy(bufs[b], accum.at[idx_win.at[1, j]],
                                add=True)
                pltpu.async_copy(hw_hbm.at[idx_win.at[0, j + NBUF]],
                                 bufs[b], sems[b])
            return carry

        lax.fori_loop(0, (SUB - NBUF) // NBUF, chunk_step, 0)

        for b in range(NBUF):
            j = SUB - NBUF + b
            wait_gather(b)
            pltpu.sync_copy(bufs[b], accum.at[idx_win.at[1, j]], add=True)

    plsc.subcore_barrier()

    pltpu.sync_copy(accum.at[pl.ds(base, ROWS_PT)],
                    out_hbm.at[c, pl.ds(base, ROWS_PT)])


def _ep_body(p_ref, b_ref, o_ref):
    o_ref[...] = jnp.maximum(p_ref[0] + p_ref[1] + b_ref[...], 0.0)


def _epilogue(partials, b):
    return pl.pallas_call(
        _ep_body,
        grid=(10,),
        in_specs=[
            pl.BlockSpec((NC, N // 10, F), lambda i: (0, i, 0)),
            pl.BlockSpec((1, F), lambda i: (0, 0)),
        ],
        out_specs=pl.BlockSpec((N // 10, F), lambda i: (i, 0)),
        out_shape=jax.ShapeDtypeStruct((N, F), jnp.float32),
    )(partials, b.reshape(1, F))


def kernel(h, edge_index, W, b):
    e = edge_index.astype(jnp.int32)
    pad = E_PAD - E
    pad_iota = lax.iota(jnp.int32, pad)
    src = jnp.concatenate([e[0], pad_iota % N])
    dst = jnp.concatenate([e[1], N + pad_iota % (N_PAD - N)])
    src = src.reshape(NW, NSUP, SUB, CH)
    dst = dst.reshape(NW, NSUP, SUB, CH)
    edges = jnp.stack([src, dst], axis=2)
    zeros = jnp.zeros((ROWS_PT, F), jnp.float32)

    hw = _matmul(h, W)
    partials = _scatter_gather(edges, hw, zeros)
    return _epilogue(partials, b)
